# Initial kernel scaffold; baseline (speedup 1.0000x reference)
import jax
import jax.numpy as jnp
from jax.experimental import pallas as pl


def kernel(x, edge_index, W_l1, W_r1, b1, W_l2, W_r2, b2):
    raise NotImplementedError("write your pallas kernel here")


# speedup vs baseline: 7.7655x; 7.7655x over previous
import functools

import jax
import jax.numpy as jnp
from jax import lax
from jax.experimental import pallas as pl
from jax.experimental.pallas import tpu as pltpu
from jax.experimental.pallas import tpu_sc as plsc

N = 10000
E = 320000
D = 128

NC = 2
NS = 16
NW = NC * NS
EPW = E // NW
CH = 80
NCH = EPW // CH
NP = 10112
RPT = NP // NS
CNP = 10240
CPT = 1280

_mesh = lambda: plsc.VectorSubcoreMesh(core_axis_name="c", subcore_axis_name="s")


def _sc_body(y, src, dst, z2, z1, s_out, cnt_out, src_v, dst_v, rows_v,
             ones_v, shared_s, shared_cnt, gsem, with_cnt):
    c = lax.axis_index("c")
    s = lax.axis_index("s")
    wid = s * NC + c

    pltpu.sync_copy(src.at[wid], src_v)
    pltpu.sync_copy(dst.at[wid], dst_v)
    pltpu.sync_copy(z2, shared_s.at[pl.ds(s * RPT, RPT)])
    if with_cnt:
        @pl.when(s < CNP // CPT)
        def _():
            pltpu.sync_copy(z1, shared_cnt.at[pl.ds(s * CPT, CPT)])
        for k in range(CH // 16):
            ones_v[pl.ds(k * 16, 16)] = jnp.full((16,), 1.0, jnp.float32)
    plsc.subcore_barrier()

    def chunk(j, carry):
        srcj = src_v.at[j]
        dstj = dst_v.at[j]
        pltpu.async_copy(y.at[srcj], rows_v, gsem).wait()
        pltpu.sync_copy(rows_v, shared_s.at[dstj], add=True)
        if with_cnt:
            pltpu.sync_copy(ones_v, shared_cnt.at[dstj], add=True)
        return carry

    lax.fori_loop(0, NCH, chunk, 0)
    plsc.subcore_barrier()

    pltpu.sync_copy(shared_s.at[pl.ds(s * RPT, RPT)],
                    s_out.at[c, pl.ds(s * RPT, RPT)])
    if with_cnt:
        @pl.when(s < CNP // CPT)
        def _():
            pltpu.sync_copy(shared_cnt.at[pl.ds(s * CPT, CPT)],
                            cnt_out.at[c, pl.ds(s * CPT, CPT)])


def _sc_agg_cnt(y, src, dst, z2, z1):
    def body(y_r, src_r, dst_r, z2_r, z1_r, s_out_r, cnt_out_r,
             src_v, dst_v, rows_v, ones_v, shared_s, shared_cnt, gsem):
        _sc_body(y_r, src_r, dst_r, z2_r, z1_r, s_out_r, cnt_out_r,
                 src_v, dst_v, rows_v, ones_v, shared_s, shared_cnt,
                 gsem, True)

    return pl.kernel(
        body,
        out_type=(jax.ShapeDtypeStruct((NC, NP, D), jnp.float32),
                  jax.ShapeDtypeStruct((NC, CNP), jnp.float32)),
        mesh=_mesh(),
        scratch_types=[
            pltpu.VMEM((NCH, CH), jnp.int32),
            pltpu.VMEM((NCH, CH), jnp.int32),
            pltpu.VMEM((CH, D), jnp.float32),
            pltpu.VMEM((CH,), jnp.float32),
            pltpu.VMEM_SHARED((NP, D), jnp.float32),
            pltpu.VMEM_SHARED((CNP,), jnp.float32),
            pltpu.SemaphoreType.DMA,
        ],
    )(y, src, dst, z2, z1)


def _sc_agg(y, src, dst, z2):
    def body(y_r, src_r, dst_r, z2_r, s_out_r,
             src_v, dst_v, rows_v, shared_s, gsem):
        _sc_body(y_r, src_r, dst_r, z2_r, None, s_out_r, None,
                 src_v, dst_v, rows_v, None, shared_s, None, gsem, False)

    return pl.kernel(
        body,
        out_type=jax.ShapeDtypeStruct((NC, NP, D), jnp.float32),
        mesh=_mesh(),
        scratch_types=[
            pltpu.VMEM((NCH, CH), jnp.int32),
            pltpu.VMEM((NCH, CH), jnp.int32),
            pltpu.VMEM((CH, D), jnp.float32),
            pltpu.VMEM_SHARED((NP, D), jnp.float32),
            pltpu.SemaphoreType.DMA,
        ],
    )(y, src, dst, z2)


ROWS_B = 2000


def _mm_body(x_r, w_r, o_r):
    o_r[...] = jnp.dot(x_r[...], w_r[...], preferred_element_type=jnp.float32)


def _tc_mm(x, w):
    return pl.pallas_call(
        _mm_body,
        grid=(N // ROWS_B,),
        in_specs=[pl.BlockSpec((ROWS_B, D), lambda i: (i, 0)),
                  pl.BlockSpec((D, D), lambda i: (0, 0))],
        out_specs=pl.BlockSpec((ROWS_B, D), lambda i: (i, 0)),
        out_shape=jax.ShapeDtypeStruct((N, D), jnp.float32),
    )(x, w)


def _tc_mid_body(s_r, ct_r, x_r, wr_r, b_r, wl2_r, h_r, y2_r):
    tot = ct_r[:, 0:1] + ct_r[:, 1:2]
    inv = 1.0 / jnp.maximum(tot, 1.0)
    agg = (s_r[0] + s_r[1]) * inv
    xw = jnp.dot(x_r[...], wr_r[...], preferred_element_type=jnp.float32)
    h = jnp.maximum(agg + xw + b_r[...], 0.0)
    h_r[...] = h
    y2_r[...] = jnp.dot(h, wl2_r[...], preferred_element_type=jnp.float32)


def _tc_mid(s1, cnt_t, x, wr, b, wl2):
    return pl.pallas_call(
        _tc_mid_body,
        grid=(N // ROWS_B,),
        in_specs=[pl.BlockSpec((NC, ROWS_B, D), lambda i: (0, i, 0)),
                  pl.BlockSpec((ROWS_B, NC), lambda i: (i, 0)),
                  pl.BlockSpec((ROWS_B, D), lambda i: (i, 0)),
                  pl.BlockSpec((D, D), lambda i: (0, 0)),
                  pl.BlockSpec((1, D), lambda i: (0, 0)),
                  pl.BlockSpec((D, D), lambda i: (0, 0))],
        out_specs=(pl.BlockSpec((ROWS_B, D), lambda i: (i, 0)),
                   pl.BlockSpec((ROWS_B, D), lambda i: (i, 0))),
        out_shape=(jax.ShapeDtypeStruct((N, D), jnp.float32),
                   jax.ShapeDtypeStruct((N, D), jnp.float32)),
    )(s1, cnt_t, x, wr, b, wl2)


def _tc_out_body(s_r, ct_r, h_r, wr_r, b_r, o_r):
    tot = ct_r[:, 0:1] + ct_r[:, 1:2]
    inv = 1.0 / jnp.maximum(tot, 1.0)
    agg = (s_r[0] + s_r[1]) * inv
    hw = jnp.dot(h_r[...], wr_r[...], preferred_element_type=jnp.float32)
    o_r[...] = agg + hw + b_r[...]


def _tc_out(s2, cnt_t, h, wr, b):
    return pl.pallas_call(
        _tc_out_body,
        grid=(N // ROWS_B,),
        in_specs=[pl.BlockSpec((NC, ROWS_B, D), lambda i: (0, i, 0)),
                  pl.BlockSpec((ROWS_B, NC), lambda i: (i, 0)),
                  pl.BlockSpec((ROWS_B, D), lambda i: (i, 0)),
                  pl.BlockSpec((D, D), lambda i: (0, 0)),
                  pl.BlockSpec((1, D), lambda i: (0, 0))],
        out_specs=pl.BlockSpec((ROWS_B, D), lambda i: (i, 0)),
        out_shape=jax.ShapeDtypeStruct((N, D), jnp.float32),
    )(s2, cnt_t, h, wr, b)


def kernel(x, edge_index, W_l1, W_r1, b1, W_l2, W_r2, b2):
    src = edge_index[0].astype(jnp.int32).reshape(NW, NCH, CH)
    dst = edge_index[1].astype(jnp.int32).reshape(NW, NCH, CH)
    z2 = jnp.zeros((RPT, D), jnp.float32)
    z1 = jnp.zeros((CPT,), jnp.float32)

    y1 = _tc_mm(x, W_l1)
    s1, cnt = _sc_agg_cnt(y1, src, dst, z2, z1)
    cnt_t = cnt.T
    h, y2 = _tc_mid(s1, cnt_t, x, W_r1, b1.reshape(1, D), W_l2)
    s2 = _sc_agg(y2, src, dst, z2)
    return _tc_out(s2, cnt_t, h, W_r2, b2.reshape(1, D))


# speedup vs baseline: 9.0796x; 1.1692x over previous
import jax
import jax.numpy as jnp
from jax import lax
from jax.experimental import pallas as pl
from jax.experimental.pallas import tpu as pltpu
from jax.experimental.pallas import tpu_sc as plsc

N = 10000
E = 320000
D = 128

NC = 2
NS = 16
NW = NC * NS
EPW = E // NW
CH = 128
NCH = 79
EPWP = NCH * CH
PAD = EPWP - EPW
NP = 10112
RPT = NP // NS
CNP = 10240
CPT = 1280

_mesh = lambda: plsc.VectorSubcoreMesh(core_axis_name="c", subcore_axis_name="s")


def _sc_agg(y, src, dst, z2):
    def body(y_r, src_r, dst_r, z2_r, s_out_r,
             src_v, dst_v, rows, shared_s, gsem):
        c = lax.axis_index("c")
        s = lax.axis_index("s")
        wid = s * NC + c

        pltpu.sync_copy(src_r.at[wid], src_v)
        pltpu.sync_copy(dst_r.at[wid], dst_v)
        pltpu.sync_copy(z2_r, shared_s.at[pl.ds(s * RPT, RPT)])
        plsc.subcore_barrier()

        def chunk(j, carry):
            pltpu.async_copy(y_r.at[src_v.at[j]], rows, gsem).wait()
            pltpu.sync_copy(rows, shared_s.at[dst_v.at[j]], add=True)
            return carry

        lax.fori_loop(0, NCH, chunk, 0)
        plsc.subcore_barrier()

        pltpu.sync_copy(shared_s.at[pl.ds(s * RPT, RPT)],
                        s_out_r.at[c, pl.ds(s * RPT, RPT)])

    return pl.kernel(
        body,
        out_type=jax.ShapeDtypeStruct((NC, NP, D), jnp.float32),
        mesh=_mesh(),
        scratch_types=[
            pltpu.VMEM((NCH, CH), jnp.int32),
            pltpu.VMEM((NCH, CH), jnp.int32),
            pltpu.VMEM((CH, D), jnp.float32),
            pltpu.VMEM_SHARED((NP, D), jnp.float32),
            pltpu.SemaphoreType.DMA,
        ],
    )(y, src, dst, z2)


def _sc_cnt(dst, z1):
    def body(dst_r, z1_r, cnt_out_r, dst_v, ones_v, shared_cnt):
        c = lax.axis_index("c")
        s = lax.axis_index("s")
        wid = s * NC + c

        pltpu.sync_copy(dst_r.at[wid], dst_v)
        @pl.when(s < CNP // CPT)
        def _():
            pltpu.sync_copy(z1_r, shared_cnt.at[pl.ds(s * CPT, CPT)])
        for k in range(CH // 16):
            ones_v[pl.ds(k * 16, 16)] = jnp.full((16,), 1.0, jnp.float32)
        plsc.subcore_barrier()

        def chunk(j, carry):
            pltpu.sync_copy(ones_v, shared_cnt.at[dst_v.at[j]], add=True)
            return carry

        lax.fori_loop(0, NCH, chunk, 0)
        plsc.subcore_barrier()

        @pl.when(s < CNP // CPT)
        def _():
            pltpu.sync_copy(shared_cnt.at[pl.ds(s * CPT, CPT)],
                            cnt_out_r.at[c, pl.ds(s * CPT, CPT)])

    return pl.kernel(
        body,
        out_type=jax.ShapeDtypeStruct((NC, CNP), jnp.float32),
        mesh=_mesh(),
        scratch_types=[
            pltpu.VMEM((NCH, CH), jnp.int32),
            pltpu.VMEM((CH,), jnp.float32),
            pltpu.VMEM_SHARED((CNP,), jnp.float32),
        ],
    )(dst, z1)


ROWS_B = 2000


def _mm_body(x_r, w_r, o_r):
    o_r[...] = jnp.dot(x_r[...], w_r[...], preferred_element_type=jnp.float32)


def _tc_mm(x, w):
    return pl.pallas_call(
        _mm_body,
        grid=(N // ROWS_B,),
        in_specs=[pl.BlockSpec((ROWS_B, D), lambda i: (i, 0)),
                  pl.BlockSpec((D, D), lambda i: (0, 0))],
        out_specs=pl.BlockSpec((ROWS_B, D), lambda i: (i, 0)),
        out_shape=jax.ShapeDtypeStruct((N, D), jnp.float32),
    )(x, w)


def _tc_mid_body(s_r, ct_r, x_r, wr_r, b_r, wl2_r, h_r, y2_r):
    tot = ct_r[:, 0:1] + ct_r[:, 1:2]
    inv = 1.0 / jnp.maximum(tot, 1.0)
    agg = (s_r[0] + s_r[1]) * inv
    xw = jnp.dot(x_r[...], wr_r[...], preferred_element_type=jnp.float32)
    h = jnp.maximum(agg + xw + b_r[...], 0.0)
    h_r[...] = h
    y2_r[...] = jnp.dot(h, wl2_r[...], preferred_element_type=jnp.float32)


def _tc_mid(s1, cnt_t, x, wr, b, wl2):
    return pl.pallas_call(
        _tc_mid_body,
        grid=(N // ROWS_B,),
        in_specs=[pl.BlockSpec((NC, ROWS_B, D), lambda i: (0, i, 0)),
                  pl.BlockSpec((ROWS_B, NC), lambda i: (i, 0)),
                  pl.BlockSpec((ROWS_B, D), lambda i: (i, 0)),
                  pl.BlockSpec((D, D), lambda i: (0, 0)),
                  pl.BlockSpec((1, D), lambda i: (0, 0)),
                  pl.BlockSpec((D, D), lambda i: (0, 0))],
        out_specs=(pl.BlockSpec((ROWS_B, D), lambda i: (i, 0)),
                   pl.BlockSpec((ROWS_B, D), lambda i: (i, 0))),
        out_shape=(jax.ShapeDtypeStruct((N, D), jnp.float32),
                   jax.ShapeDtypeStruct((N, D), jnp.float32)),
    )(s1, cnt_t, x, wr, b, wl2)


def _tc_out_body(s_r, ct_r, h_r, wr_r, b_r, o_r):
    tot = ct_r[:, 0:1] + ct_r[:, 1:2]
    inv = 1.0 / jnp.maximum(tot, 1.0)
    agg = (s_r[0] + s_r[1]) * inv
    hw = jnp.dot(h_r[...], wr_r[...], preferred_element_type=jnp.float32)
    o_r[...] = agg + hw + b_r[...]


def _tc_out(s2, cnt_t, h, wr, b):
    return pl.pallas_call(
        _tc_out_body,
        grid=(N // ROWS_B,),
        in_specs=[pl.BlockSpec((NC, ROWS_B, D), lambda i: (0, i, 0)),
                  pl.BlockSpec((ROWS_B, NC), lambda i: (i, 0)),
                  pl.BlockSpec((ROWS_B, D), lambda i: (i, 0)),
                  pl.BlockSpec((D, D), lambda i: (0, 0)),
                  pl.BlockSpec((1, D), lambda i: (0, 0))],
        out_specs=pl.BlockSpec((ROWS_B, D), lambda i: (i, 0)),
        out_shape=jax.ShapeDtypeStruct((N, D), jnp.float32),
    )(s2, cnt_t, h, wr, b)


def _pad_edges(idx, pad_base):
    tiles = idx.astype(jnp.int32).reshape(NW, EPW)
    padv = pad_base + jnp.arange(PAD, dtype=jnp.int32)
    pad = jnp.broadcast_to(padv, (NW, PAD))
    return jnp.concatenate([tiles, pad], axis=1).reshape(NW, NCH, CH)


def kernel(x, edge_index, W_l1, W_r1, b1, W_l2, W_r2, b2):
    src = _pad_edges(edge_index[0], 0)
    dst = _pad_edges(edge_index[1], N)
    z2 = jnp.zeros((RPT, D), jnp.float32)
    z1 = jnp.zeros((CPT,), jnp.float32)

    cnt = _sc_cnt(dst, z1)
    cnt_t = cnt.T
    y1 = _tc_mm(x, W_l1)
    s1 = _sc_agg(y1, src, dst, z2)
    h, y2 = _tc_mid(s1, cnt_t, x, W_r1, b1.reshape(1, D), W_l2)
    s2 = _sc_agg(y2, src, dst, z2)
    return _tc_out(s2, cnt_t, h, W_r2, b2.reshape(1, D))
